# baseline (speedup 1.0000x reference)
import functools

import jax
import jax.numpy as jnp
from jax import lax
from jax.experimental import pallas as pl
from jax.experimental.pallas import tpu as pltpu
from jax.experimental.pallas import tpu_sc as plsc

K = 1024
L = 16
NC, NS = 2, 16
NW = NC * NS


def _make_sc_call(n_rows, hw, n_batch):
    rows_per_w = n_rows // NW
    rows_per_b = n_rows // n_batch
    n_chunks = hw // L
    unroll = 8

    mesh = plsc.VectorSubcoreMesh(
        core_axis_name="c", subcore_axis_name="s",
        num_cores=NC, num_subcores=NS)

    @functools.partial(
        pl.kernel,
        out_type=jax.ShapeDtypeStruct((n_rows, K), jnp.float32),
        mesh=mesh,
        compiler_params=pltpu.CompilerParams(needs_layout_passes=False),
        scratch_types=[
            pltpu.VMEM((hw,), jnp.int32),
            pltpu.VMEM((hw,), jnp.float32),
            pltpu.VMEM((L * K,), jnp.float32),
            pltpu.VMEM((K,), jnp.float32),
        ],
    )
    def sc_call(img_hbm, seg_hbm, out_hbm, seg_v, val_v, acc_v, out_v):
        cid = lax.axis_index("c")
        sid = lax.axis_index("s")
        wid = sid * NC + cid
        b = (wid * rows_per_w) // rows_per_b
        pltpu.sync_copy(seg_hbm.at[b], seg_v)

        iot = lax.iota(jnp.int32, L)
        lane_base = iot * K
        neg = jnp.full((L,), -jnp.inf, jnp.float32)

        def do_row(r, carry):
            row = wid * rows_per_w + r
            pltpu.sync_copy(img_hbm.at[row], val_v)

            def init(j, c):
                off = pl.multiple_of(j * (L * L), L)
                for t in range(L):
                    acc_v[pl.ds(off + t * L, L)] = neg
                return c
            lax.fori_loop(0, (L * K) // (L * L), init, 0)

            def chunk(i, c):
                base = pl.multiple_of(i * (L * unroll), L)
                for u in range(unroll):
                    off = base + u * L
                    idx = seg_v[pl.ds(off, L)]
                    vals = val_v[pl.ds(off, L)]
                    addr = idx + lane_base
                    cur = plsc.load_gather(acc_v, [addr])
                    plsc.store_scatter(acc_v, [addr], jnp.maximum(cur, vals))
                return c
            lax.fori_loop(0, n_chunks // unroll, chunk, 0)

            def fin(j, c):
                off = pl.multiple_of(j * L, L)
                m = acc_v[pl.ds(off, L)]
                for l in range(1, L):
                    m = jnp.maximum(m, acc_v[pl.ds(l * K + off, L)])
                out_v[pl.ds(off, L)] = m
                return c
            lax.fori_loop(0, K // L, fin, 0)

            pltpu.sync_copy(out_v, out_hbm.at[row])
            return carry
        lax.fori_loop(0, rows_per_w, do_row, 0)

    return sc_call


def kernel(img, spx):
    B, C, H, W = img.shape
    hw = H * W
    img2 = img.reshape(B * C, hw)
    spx2 = spx.reshape(B, hw).astype(jnp.int32)
    out = _make_sc_call(B * C, hw, B)(img2, spx2)
    return out.reshape(B, C, K)


# speedup vs baseline: 1.9152x; 1.9152x over previous
import functools

import jax
import jax.numpy as jnp
from jax import lax
from jax.experimental import pallas as pl
from jax.experimental.pallas import tpu as pltpu
from jax.experimental.pallas import tpu_sc as plsc

K = 1024
L = 16
NC, NS = 2, 16
NW = NC * NS
CG = 4
CH = 3136
U = 4


def _make_sc_call(n_rows, hw, n_batch):
    rows_per_w = n_rows // NW
    rows_per_b = n_rows // n_batch
    n_dma = hw // CH
    cpg = CH // L
    n_pass = rows_per_w // CG

    mesh = plsc.VectorSubcoreMesh(
        core_axis_name="c", subcore_axis_name="s",
        num_cores=NC, num_subcores=NS)

    @functools.partial(
        pl.kernel,
        out_type=jax.ShapeDtypeStruct((n_rows * K,), jnp.float32),
        mesh=mesh,
        compiler_params=pltpu.CompilerParams(needs_layout_passes=False),
        scratch_types=(
            [pltpu.VMEM((hw,), jnp.int16)]
            + [pltpu.VMEM((L * K,), jnp.float32)] * CG
            + [pltpu.VMEM((CH,), jnp.float32)] * (2 * CG)
            + [pltpu.VMEM((K,), jnp.float32)] * CG
            + [pltpu.SemaphoreType.DMA] * (2 * CG)
        ),
    )
    def sc_call(img_hbm, seg_hbm, out_hbm, *scratch):
        seg_v = scratch[0]
        accs = scratch[1:1 + CG]
        bufs = [scratch[1 + CG + 2 * ch: 3 + CG + 2 * ch] for ch in range(CG)]
        outs = scratch[1 + 3 * CG: 1 + 4 * CG]
        sems = [scratch[1 + 4 * CG + 2 * ch: 3 + 4 * CG + 2 * ch]
                for ch in range(CG)]

        cid = lax.axis_index("c")
        sid = lax.axis_index("s")
        wid = sid * NC + cid
        b = (wid * rows_per_w) // rows_per_b
        pltpu.sync_copy(
            seg_hbm.at[pl.ds(pl.multiple_of(b * hw, 16), hw)], seg_v)

        iot = lax.iota(jnp.int32, L)
        lane_base = iot * K
        neg = jnp.full((L,), -jnp.inf, jnp.float32)

        def copy(ch, row, d, par):
            off = pl.multiple_of(row * hw + d * CH, 8)
            return pltpu.make_async_copy(
                img_hbm.at[pl.ds(off, CH)], bufs[ch][par],
                sems[ch][par])

        def initf(j, c):
            off = pl.multiple_of(j * (L * L), L)
            for t in range(L):
                for ch in range(CG):
                    accs[ch][pl.ds(off + t * L, L)] = neg
            return c
        lax.fori_loop(0, (L * K) // (L * L), initf, 0)

        def do_pass(p, carry):
            base = wid * rows_per_w + p * CG
            for ch in range(CG):
                copy(ch, base + ch, 0, 0).start()
                copy(ch, base + ch, 1, 1).start()

            def dstep(dd, c):
                for par in (0, 1):
                    d = dd * 2 + par
                    for ch in range(CG):
                        copy(ch, base + ch, d, par).wait()

                    def grp(g, c2):
                        def loads(t):
                            po = pl.multiple_of(
                                d * CH + g * (U * L) + t * (2 * L), 2 * L)
                            lo = pl.multiple_of(
                                g * (U * L) + t * (2 * L), 2 * L)
                            raw = seg_v[pl.ds(po, 2 * L)]
                            i0, i1 = plsc.unpack(
                                raw, format=plsc.PackFormat.INTERLEAVED,
                                preferred_element_type=jnp.int32)
                            a0 = i0 + lane_base
                            a1 = i1 + lane_base
                            v0 = [bufs[ch][par][pl.ds(lo, L)]
                                  for ch in range(CG)]
                            v1 = [bufs[ch][par][pl.ds(lo + L, L)]
                                  for ch in range(CG)]
                            return a0, a1, v0, v1
                        cur = loads(0)
                        n_pair = U // 2
                        for t in range(n_pair):
                            a0, a1, v0, v1 = cur
                            g0 = [plsc.load_gather(accs[ch], [a0])
                                  for ch in range(CG)]
                            if t + 1 < n_pair:
                                nxt = loads(t + 1)
                            for ch in range(CG):
                                plsc.store_scatter(
                                    accs[ch], [a0],
                                    jnp.maximum(g0[ch], v0[ch]))
                            g1 = [plsc.load_gather(accs[ch], [a1])
                                  for ch in range(CG)]
                            for ch in range(CG):
                                plsc.store_scatter(
                                    accs[ch], [a1],
                                    jnp.maximum(g1[ch], v1[ch]))
                            if t + 1 < n_pair:
                                cur = nxt
                        return c2
                    lax.fori_loop(0, cpg // U, grp, 0)

                    nd = d + 2

                    @pl.when(nd < n_dma)
                    def _():
                        for ch in range(CG):
                            copy(ch, base + ch, nd, par).start()
                return c
            lax.fori_loop(0, n_dma // 2, dstep, 0)

            def fin(j, c):
                off = pl.multiple_of(j * L, L)
                for ch in range(CG):
                    m = accs[ch][pl.ds(off, L)]
                    for l in range(1, L):
                        m = jnp.maximum(m, accs[ch][pl.ds(l * K + off, L)])
                    outs[ch][pl.ds(off, L)] = m
                    for l in range(L):
                        accs[ch][pl.ds(l * K + off, L)] = neg
                return c
            lax.fori_loop(0, K // L, fin, 0)

            for ch in range(CG):
                o_off = pl.multiple_of((base + ch) * K, 8)
                pltpu.sync_copy(outs[ch], out_hbm.at[pl.ds(o_off, K)])
            return carry
        lax.fori_loop(0, n_pass, do_pass, 0)

    return sc_call


def kernel(img, spx):
    B, C, H, W = img.shape
    hw = H * W
    img2 = img.reshape(B * C * hw)
    spx2 = (spx.reshape(B * hw // 32, 2, 16)
            .transpose(0, 2, 1).reshape(B * hw).astype(jnp.int16))
    out = _make_sc_call(B * C, hw, B)(img2, spx2)
    return out.reshape(B, C, K)


# speedup vs baseline: 2.0367x; 1.0635x over previous
import functools

import jax
import jax.numpy as jnp
from jax import lax
from jax.experimental import pallas as pl
from jax.experimental.pallas import tpu as pltpu
from jax.experimental.pallas import tpu_sc as plsc

K = 1024
L = 16
NC, NS = 2, 16
NW = NC * NS
CG = 4
CH = 3136
U = 4


def _make_sc_call(n_rows, hw, n_batch):
    rows_per_w = n_rows // NW
    rows_per_b = n_rows // n_batch
    n_dma = hw // CH
    cpg = CH // L
    n_pass = rows_per_w // CG

    mesh = plsc.VectorSubcoreMesh(
        core_axis_name="c", subcore_axis_name="s",
        num_cores=NC, num_subcores=NS)

    @functools.partial(
        pl.kernel,
        out_type=jax.ShapeDtypeStruct((n_rows * K,), jnp.float32),
        mesh=mesh,
        compiler_params=pltpu.CompilerParams(needs_layout_passes=False),
        scratch_types=(
            [pltpu.VMEM((hw // 2,), jnp.int32)]
            + [pltpu.VMEM((L * K,), jnp.float32)] * CG
            + [pltpu.VMEM((CH,), jnp.float32)] * (2 * CG)
            + [pltpu.VMEM((K,), jnp.float32)] * CG
            + [pltpu.SemaphoreType.DMA] * (2 * CG)
        ),
    )
    def sc_call(img_hbm, seg_hbm, out_hbm, *scratch):
        seg_v = scratch[0]
        accs = scratch[1:1 + CG]
        bufs = [scratch[1 + CG + 2 * ch: 3 + CG + 2 * ch] for ch in range(CG)]
        outs = scratch[1 + 3 * CG: 1 + 4 * CG]
        sems = [scratch[1 + 4 * CG + 2 * ch: 3 + 4 * CG + 2 * ch]
                for ch in range(CG)]

        cid = lax.axis_index("c")
        sid = lax.axis_index("s")
        wid = sid * NC + cid
        b = (wid * rows_per_w) // rows_per_b
        pltpu.sync_copy(
            seg_hbm.at[pl.ds(pl.multiple_of(b * (hw // 2), 8), hw // 2)],
            seg_v)

        iot = lax.iota(jnp.int32, L)
        lane_base = iot * K
        neg = jnp.full((L,), -jnp.inf, jnp.float32)

        def copy(ch, row, d, par):
            off = pl.multiple_of(row * hw + d * CH, 8)
            return pltpu.make_async_copy(
                img_hbm.at[pl.ds(off, CH)], bufs[ch][par],
                sems[ch][par])

        def initf(j, c):
            off = pl.multiple_of(j * (L * L), L)
            for t in range(L):
                for ch in range(CG):
                    accs[ch][pl.ds(off + t * L, L)] = neg
            return c
        lax.fori_loop(0, (L * K) // (L * L), initf, 0)

        def do_pass(p, carry):
            base = wid * rows_per_w + p * CG
            for ch in range(CG):
                copy(ch, base + ch, 0, 0).start()
                copy(ch, base + ch, 1, 1).start()

            def dstep(dd, c):
                for par in (0, 1):
                    d = dd * 2 + par
                    for ch in range(CG):
                        copy(ch, base + ch, d, par).wait()

                    def grp(g, c2):
                        def loads(t):
                            po = pl.multiple_of(
                                d * CH + g * (U * L) + t * (2 * L), 2 * L)
                            lo = pl.multiple_of(
                                g * (U * L) + t * (2 * L), 2 * L)
                            raw = seg_v[pl.ds(po // 2, L)]
                            i0 = lax.bitwise_and(raw, jnp.int32(0xFFFF))
                            i1 = lax.shift_right_logical(raw, jnp.int32(16))
                            a0 = i0 + lane_base
                            a1 = i1 + lane_base
                            v0 = [bufs[ch][par][pl.ds(lo, L)]
                                  for ch in range(CG)]
                            v1 = [bufs[ch][par][pl.ds(lo + L, L)]
                                  for ch in range(CG)]
                            return a0, a1, v0, v1
                        cur = loads(0)
                        n_pair = U // 2
                        for t in range(n_pair):
                            a0, a1, v0, v1 = cur
                            g0 = [plsc.load_gather(accs[ch], [a0])
                                  for ch in range(CG)]
                            if t + 1 < n_pair:
                                nxt = loads(t + 1)
                            for ch in range(CG):
                                plsc.store_scatter(
                                    accs[ch], [a0],
                                    jnp.maximum(g0[ch], v0[ch]))
                            g1 = [plsc.load_gather(accs[ch], [a1])
                                  for ch in range(CG)]
                            for ch in range(CG):
                                plsc.store_scatter(
                                    accs[ch], [a1],
                                    jnp.maximum(g1[ch], v1[ch]))
                            if t + 1 < n_pair:
                                cur = nxt
                        return c2
                    lax.fori_loop(0, cpg // U, grp, 0)

                    nd = d + 2

                    @pl.when(nd < n_dma)
                    def _():
                        for ch in range(CG):
                            copy(ch, base + ch, nd, par).start()
                return c
            lax.fori_loop(0, n_dma // 2, dstep, 0)

            def fin(j, c):
                off = pl.multiple_of(j * L, L)
                for ch in range(CG):
                    m = accs[ch][pl.ds(off, L)]
                    for l in range(1, L):
                        m = jnp.maximum(m, accs[ch][pl.ds(l * K + off, L)])
                    outs[ch][pl.ds(off, L)] = m
                    for l in range(L):
                        accs[ch][pl.ds(l * K + off, L)] = neg
                return c
            lax.fori_loop(0, K // L, fin, 0)

            for ch in range(CG):
                o_off = pl.multiple_of((base + ch) * K, 8)
                pltpu.sync_copy(outs[ch], out_hbm.at[pl.ds(o_off, K)])
            return carry
        lax.fori_loop(0, n_pass, do_pass, 0)

    return sc_call


def kernel(img, spx):
    B, C, H, W = img.shape
    hw = H * W
    img2 = img.reshape(B * C * hw)
    s = spx.reshape(B * hw // 32, 2, 16).astype(jnp.int32)
    spx2 = (s[:, 0, :] | (s[:, 1, :] << 16)).reshape(B * hw // 2)
    out = _make_sc_call(B * C, hw, B)(img2, spx2)
    return out.reshape(B, C, K)
